# baseline (speedup 1.0000x reference)
import jax
import jax.numpy as jnp
import numpy as np
from jax import lax
from jax.experimental import pallas as pl
from jax.experimental.pallas import tpu as pltpu
from jax.experimental.pallas import tpu_sc as plsc

N = 10000
E = 320000
H = 128
HEADS = 4
DH = 32
T = 2
SCALE = float(np.sqrt(DH))

NC = 2
NS = 16
NW = NC * NS
EPW = E // NW
B = 32
NB = EPW // B
TAIL = EPW - NB * B
SB = 4
CB = 2 * B

SBASE = 10240
AROWS = SBASE + 320
DUMP = SBASE + 316
ZR = 664
ZR_LAST = AROWS - 15 * ZR

NPAD = 320 * DH

_f32 = jnp.float32


def _prep_body(x_ref, wq_ref, bq_ref, wk_ref, bk_ref, wv_ref, bv_ref,
               qt_ref, kv_ref):
    xb = x_ref[...]
    q = jnp.dot(xb, wq_ref[...].T, preferred_element_type=_f32) + bq_ref[...]
    k = jnp.dot(xb, wk_ref[...].T, preferred_element_type=_f32) + bk_ref[...]
    v = jnp.dot(xb, wv_ref[...].T, preferred_element_type=_f32) + bv_ref[...]
    qt_ref[...] = q * (1.0 / SCALE)
    kv_ref[...] = jnp.concatenate([k, v], axis=1)


def _prep(x, Wq, bq, Wk, bk, Wv, bv):
    rows = N // 10
    return pl.pallas_call(
        _prep_body,
        grid=(10,),
        in_specs=[
            pl.BlockSpec((rows, H), lambda i: (i, 0)),
            pl.BlockSpec((H, H), lambda i: (0, 0)),
            pl.BlockSpec((H,), lambda i: (0,)),
            pl.BlockSpec((H, H), lambda i: (0, 0)),
            pl.BlockSpec((H,), lambda i: (0,)),
            pl.BlockSpec((H, H), lambda i: (0, 0)),
            pl.BlockSpec((H,), lambda i: (0,)),
        ],
        out_specs=[
            pl.BlockSpec((rows, H), lambda i: (i, 0)),
            pl.BlockSpec((rows, 2 * H), lambda i: (i, 0)),
        ],
        out_shape=[
            jax.ShapeDtypeStruct((N, H), _f32),
            jax.ShapeDtypeStruct((N, 2 * H), _f32),
        ],
    )(x, Wq, bq, Wk, bk, Wv, bv)


def _bcast_gather(x, idx):
    dn = lax.GatherDimensionNumbers(
        offset_dims=(), collapsed_slice_dims=(0,), start_index_map=(0,))
    return lax.gather(x, idx[:, None], dn, slice_sizes=(1,),
                      mode=lax.GatherScatterMode.PROMISE_IN_BOUNDS)


def _edge_kernel_body(qt_hbm, kv_hbm, src_hbm, dst_hbm, p_hbm, sk_hbm, sv_hbm,
                      out_hbm,
                      srcv, dstv, pv, idxc, skv, svv,
                      qtv, kvv, cvb,
                      acc, seml0, seml1, semg, sems):
    cid = lax.axis_index("c")
    sid = lax.axis_index("s")
    wid = sid * NC + cid
    lane = lax.iota(jnp.int32, 16)
    zero16 = jnp.zeros((16,), _f32)

    pltpu.sync_copy(sk_hbm, skv)
    pltpu.sync_copy(sv_hbm, svv)
    skc = [skv[pl.ds(c * 16, 16)] for c in range(8)]
    svc = [svv[pl.ds(c * 16, 16)] for c in range(8)]

    def _zrow(i, _):
        cvb[i // 8, pl.ds((i % 8) * 16, 16)] = zero16
        return 0
    lax.fori_loop(0, CB * (H // 16), _zrow, 0)

    z0 = sid * ZR

    @pl.when(sid == NS - 1)
    def _():
        for j in range(ZR_LAST // CB):
            pltpu.sync_copy(cvb, acc.at[pl.ds(z0 + j * CB, CB)])
        pltpu.sync_copy(cvb.at[pl.ds(0, ZR_LAST % CB)],
                        acc.at[pl.ds(z0 + (ZR_LAST // CB) * CB,
                                     ZR_LAST % CB)])

    @pl.when(sid != NS - 1)
    def _():
        for j in range(ZR // CB):
            pltpu.sync_copy(cvb, acc.at[pl.ds(z0 + j * CB, CB)])
        pltpu.sync_copy(cvb.at[pl.ds(0, ZR % CB)],
                        acc.at[pl.ds(z0 + (ZR // CB) * CB, ZR % CB)])

    plsc.subcore_barrier()

    lx8 = jnp.bitwise_xor(lane, 8)
    lx4 = jnp.bitwise_xor(lane, 4)
    lx2 = jnp.bitwise_xor(lane, 2)
    lx1 = jnp.bitwise_xor(lane, 1)
    lane4 = (lane * 4) & 15
    laneh4 = jnp.where(lane < HEADS, lane, 0)
    lmasks = [lane // 4 == hh for hh in range(HEADS)]
    exbi = [jnp.full((16,), 4 * hh, jnp.int32) for hh in range(HEADS)]

    def _group(gslot, sbslot, boff, g):
        pch = pv[sbslot, pl.ds(boff + g * 16, 16)]
        dch = dstv[sbslot, pl.ds(boff + g * 16, 16)]

        def _one(i, e):
            sub = jnp.full((16,), i, jnp.int32)
            pe = _bcast_gather(pch, sub)
            dste = _bcast_gather(dch, sub)
            us = []
            for hh in range(HEADS):
                c0 = hh * DH
                kj0 = kvv[gslot, e, pl.ds(c0, 16)] + pe * skc[2 * hh]
                kj1 = kvv[gslot, e, pl.ds(c0 + 16, 16)] + pe * skc[2 * hh + 1]
                s = (qtv[gslot, e, pl.ds(c0, 16)] * kj0
                     + qtv[gslot, e, pl.ds(c0 + 16, 16)] * kj1)
                t = s + _bcast_gather(s, lx8)
                us.append(t + _bcast_gather(t, lx4))
            w = jnp.where(lmasks[0], us[0],
                          jnp.where(lmasks[1], us[1],
                                    jnp.where(lmasks[2], us[2], us[3])))
            x1 = w + _bcast_gather(w, lx2)
            alpha = x1 + _bcast_gather(x1, lx1)
            exr = jnp.exp(alpha)
            exl = _bcast_gather(exr, lane4)
            for j in range(H // 16):
                cvb[B + e, pl.ds(j * 16, 16)] = zero16
            colv = (dste % 32) * HEADS + laneh4
            plsc.store_scatter(cvb, [jnp.full((16,), B + e, jnp.int32), colv],
                               exl, mask=lane < HEADS)
            for hh in range(HEADS):
                exb = _bcast_gather(exr, exbi[hh])
                c0 = hh * DH
                vj0 = kvv[gslot, e, pl.ds(H + c0, 16)] + pe * svc[2 * hh]
                vj1 = kvv[gslot, e, pl.ds(H + c0 + 16, 16)] + pe * svc[2 * hh + 1]
                cvb[e, pl.ds(c0, 16)] = vj0 * exb
                cvb[e, pl.ds(c0 + 16, 16)] = vj1 * exb

        def _e(i, _):
            _one(i, g * 16 + i)
            _one(i + 8, g * 16 + i + 8)
            return 0
        lax.fori_loop(0, 8, _e, 0)
        return 0

    def _issue_loads(sbslot, sb, sem):
        base = wid * EPW + sb * SB * B
        pltpu.async_copy(src_hbm.at[pl.ds(base, SB * B)], srcv.at[sbslot],
                         sem)
        pltpu.async_copy(dst_hbm.at[pl.ds(base, SB * B)], dstv.at[sbslot],
                         sem)
        pltpu.async_copy(p_hbm.at[pl.ds(base, SB * B)], pv.at[sbslot], sem)

    def _wait_loads(sem):
        pltpu.make_async_copy(src_hbm.at[pl.ds(0, SB * B)], srcv.at[0],
                              sem).wait()
        pltpu.make_async_copy(dst_hbm.at[pl.ds(0, SB * B)], dstv.at[0],
                              sem).wait()
        pltpu.make_async_copy(p_hbm.at[pl.ds(0, SB * B)], pv.at[0],
                              sem).wait()

    def _mk_idxc(sbslot, boff):
        for j in range(B // 16):
            d = dstv[sbslot, pl.ds(boff + j * 16, 16)]
            idxc[pl.ds(j * 16, 16)] = d
            idxc[pl.ds(B + j * 16, 16)] = jnp.minimum(d // 32 + SBASE, DUMP)

    def _issue_gathers(gslot, sbslot, boff):
        pltpu.async_copy(
            kv_hbm.at[srcv.at[sbslot, pl.ds(boff, B)]], kvv.at[gslot], semg)
        pltpu.async_copy(
            qt_hbm.at[dstv.at[sbslot, pl.ds(boff, B)]], qtv.at[gslot], semg)

    def _wait_gathers():
        pltpu.make_async_copy(kv_hbm.at[srcv.at[0, pl.ds(0, B)]], kvv.at[0],
                              semg).wait()
        pltpu.make_async_copy(qt_hbm.at[dstv.at[0, pl.ds(0, B)]], qtv.at[0],
                              semg).wait()

    def _issue_scatter():
        pltpu.async_copy(cvb, acc.at[idxc], sems, add=True)

    def _wait_scatter():
        pltpu.make_async_copy(cvb, acc.at[idxc], sems).wait()

    _issue_loads(0, 0, seml0)
    _wait_loads(seml0)
    _issue_gathers(0, 0, 0)
    _issue_loads(1, 1, seml1)

    def _blk(blk, _):
        gslot = blk % 2
        ngslot = (blk + 1) % 2
        sb = blk // SB
        sbslot = sb % 2
        boff = (blk % SB) * B
        nblk = blk + 1
        nsb = nblk // SB
        nsbslot = nsb % 2
        nboff = (nblk % SB) * B

        @pl.when(blk > 0)
        def _():
            _wait_scatter()
        _mk_idxc(sbslot, boff)

        @pl.when((boff == 0) & (blk > 0) & (blk + SB < NB))
        def _():
            @pl.when(sbslot == 0)
            def _():
                _issue_loads(1, sb + 1, seml1)

            @pl.when(sbslot == 1)
            def _():
                _issue_loads(0, sb + 1, seml0)

        _wait_gathers()

        @pl.when(nblk < NB)
        def _():
            @pl.when((nboff == 0) & (nsbslot == 0))
            def _():
                _wait_loads(seml0)

            @pl.when((nboff == 0) & (nsbslot == 1))
            def _():
                _wait_loads(seml1)
            _issue_gathers(ngslot, nsbslot, nboff)

        lax.fori_loop(0, B // 16,
                      lambda g, c: _group(gslot, sbslot, boff, g), 0)
        _issue_scatter()
        return 0
    lax.fori_loop(0, NB, _blk, 0)
    _wait_scatter()

    def _ztail(i, _):
        cvb[TAIL + i // 8, pl.ds((i % 8) * 16, 16)] = zero16
        cvb[B + TAIL + i // 8, pl.ds((i % 8) * 16, 16)] = zero16
        return 0
    lax.fori_loop(0, (B - TAIL) * (H // 16), _ztail, 0)
    for j in range(B // 16):
        dstv[0, pl.ds(j * 16, 16)] = jnp.full((16,), DUMP, jnp.int32)
        srcv[0, pl.ds(j * 16, 16)] = jnp.zeros((16,), jnp.int32)
    tb = wid * EPW + NB * B
    pltpu.sync_copy(src_hbm.at[pl.ds(tb, TAIL)], srcv.at[0, pl.ds(0, TAIL)])
    pltpu.sync_copy(dst_hbm.at[pl.ds(tb, TAIL)], dstv.at[0, pl.ds(0, TAIL)])
    pltpu.sync_copy(p_hbm.at[pl.ds(tb, TAIL)], pv.at[0, pl.ds(0, TAIL)])
    _mk_idxc(0, 0)
    _issue_gathers(0, 0, 0)
    _wait_gathers()
    _group(0, 0, 0, 0)
    _issue_scatter()
    _wait_scatter()

    plsc.subcore_barrier()

    @pl.when(sid == NS - 1)
    def _():
        pltpu.sync_copy(acc.at[pl.ds(z0, ZR_LAST)],
                        out_hbm.at[cid, pl.ds(z0, ZR_LAST)])

    @pl.when(sid != NS - 1)
    def _():
        pltpu.sync_copy(acc.at[pl.ds(z0, ZR)],
                        out_hbm.at[cid, pl.ds(z0, ZR)])


def _edge(qt, kv, src, dst, p, sk, sv):
    mesh = plsc.VectorSubcoreMesh(core_axis_name="c", subcore_axis_name="s")
    fn = pl.kernel(
        _edge_kernel_body,
        out_type=jax.ShapeDtypeStruct((NC, AROWS, H), _f32),
        mesh=mesh,
        compiler_params=pltpu.CompilerParams(needs_layout_passes=False),
        scratch_types=[
            pltpu.VMEM((2, SB * B), jnp.int32),
            pltpu.VMEM((2, SB * B), jnp.int32),
            pltpu.VMEM((2, SB * B), _f32),
            pltpu.VMEM((CB,), jnp.int32),
            pltpu.VMEM((H,), _f32),
            pltpu.VMEM((H,), _f32),
            pltpu.VMEM((2, B, H), _f32),
            pltpu.VMEM((2, B, 2 * H), _f32),
            pltpu.VMEM((CB, H), _f32),
            pltpu.VMEM_SHARED((AROWS, H), _f32),
            pltpu.SemaphoreType.DMA,
            pltpu.SemaphoreType.DMA,
            pltpu.SemaphoreType.DMA,
            pltpu.SemaphoreType.DMA,
        ],
    )
    return fn(qt, kv, src, dst, p, sk, sv)


def _expand_body(stats_ref, out_ref):
    dsum = stats_ref[0] + stats_ref[1]
    jj = lax.broadcasted_iota(jnp.int32, (H, DH, H), 0)
    kk = lax.broadcasted_iota(jnp.int32, (H, DH, H), 1)
    cc = lax.broadcasted_iota(jnp.int32, (H, DH, H), 2)
    p = (jj == HEADS * kk + cc // DH).astype(_f32)
    full = lax.dot_general(dsum, p, (((1,), (0,)), ((), ())),
                           preferred_element_type=_f32)
    out_ref[...] = full.reshape(NPAD, H)


def _expand_den(parts):
    return pl.pallas_call(
        _expand_body,
        grid=(1,),
        in_specs=[pl.BlockSpec((NC, 320, H), lambda i: (0, SBASE // 320, 0))],
        out_specs=pl.BlockSpec((NPAD, H), lambda i: (0, 0)),
        out_shape=jax.ShapeDtypeStruct((NPAD, H), _f32),
    )(parts)


def _ln(x, g, b, eps=1e-12):
    u = x.mean(-1, keepdims=True)
    s = ((x - u) ** 2).mean(-1, keepdims=True)
    return g * (x - u) / jnp.sqrt(s + eps) + b


def _post_body(parts_ref, dens_ref, x_ref, h_ref, wao_ref, bao_ref, g1_ref,
               b1_ref, wi_ref, bi_ref, wo_ref, bo_ref, g2_ref, b2_ref,
               wih_ref, whh_ref, bih_ref, bhh_ref, g3_ref, b3_ref,
               xo_ref, ho_ref):
    aggv = parts_ref[0] + parts_ref[1]
    den = dens_ref[...] + 1e-16
    agg = aggv / den
    x = x_ref[...]
    h = h_ref[...]
    attn = _ln(jnp.dot(agg, wao_ref[...].T, preferred_element_type=_f32)
               + bao_ref[...] + x, g1_ref[...], b1_ref[...])
    inter = jax.nn.gelu(jnp.dot(attn, wi_ref[...].T,
                                preferred_element_type=_f32) + bi_ref[...])
    m = _ln(jnp.dot(inter, wo_ref[...].T, preferred_element_type=_f32)
            + bo_ref[...] + attn, g2_ref[...], b2_ref[...])
    gi = jnp.dot(m, wih_ref[...].T, preferred_element_type=_f32) + bih_ref[...]
    gh = jnp.dot(h, whh_ref[...].T, preferred_element_type=_f32) + bhh_ref[...]
    r = jax.nn.sigmoid(gi[:, 0:H] + gh[:, 0:H])
    z = jax.nn.sigmoid(gi[:, H:2 * H] + gh[:, H:2 * H])
    n = jnp.tanh(gi[:, 2 * H:3 * H] + r * gh[:, 2 * H:3 * H])
    hn = (1.0 - z) * n + z * h
    ho_ref[...] = hn
    xo_ref[...] = _ln(hn, g3_ref[...], b3_ref[...])


def _post(parts, dens, x, h, Wao, bao, g1, b1, Wi, bi, Wo, bo, g2, b2,
          W_ih, W_hh, b_ih, b_hh, g3, b3):
    rows = N // 10
    full = lambda shape: pl.BlockSpec(shape, lambda i: (0,) * len(shape))
    vec = full((H,))
    return pl.pallas_call(
        _post_body,
        grid=(10,),
        in_specs=[
            pl.BlockSpec((NC, rows, H), lambda i: (0, i, 0)),
            pl.BlockSpec((rows, H), lambda i: (i, 0)),
            pl.BlockSpec((rows, H), lambda i: (i, 0)),
            pl.BlockSpec((rows, H), lambda i: (i, 0)),
            full((H, H)), vec,
            vec, vec,
            full((4 * H, H)), full((4 * H,)),
            full((H, 4 * H)), vec,
            vec, vec,
            full((3 * H, H)), full((3 * H, H)),
            full((3 * H,)), full((3 * H,)),
            vec, vec,
        ],
        out_specs=[
            pl.BlockSpec((rows, H), lambda i: (i, 0)),
            pl.BlockSpec((rows, H), lambda i: (i, 0)),
        ],
        out_shape=[
            jax.ShapeDtypeStruct((N, H), _f32),
            jax.ShapeDtypeStruct((N, H), _f32),
        ],
    )(parts, dens, x, h, Wao, bao, g1, b1, Wi, bi, Wo, bo, g2, b2,
      W_ih, W_hh, b_ih, b_hh, g3, b3)


def kernel(x, edge_index, edge_attr, Wq, bq, Wk, bk, Wv, bv, Wao, bao, g1, b1,
           Wi, bi, Wo, bo, g2, b2, W_ih, W_hh, b_ih, b_hh, g3, b3):
    src = edge_index[0]
    dst = edge_index[1]
    sk = jnp.sum(Wk, axis=1)
    sv = jnp.sum(Wv, axis=1)
    h = x
    for _ in range(T):
        qt, kv = _prep(x, Wq, bq, Wk, bk, Wv, bv)
        parts = _edge(qt, kv, src, dst, edge_attr, sk, sv)
        dens = _expand_den(parts)
        x, h = _post(parts, dens, x, h, Wao, bao, g1, b1, Wi, bi, Wo, bo,
                     g2, b2, W_ih, W_hh, b_ih, b_hh, g3, b3)
    return x
